# Initial kernel scaffold; baseline (speedup 1.0000x reference)
import jax
import jax.numpy as jnp
from jax.experimental import pallas as pl


def kernel(input_ids, weights):
    raise NotImplementedError("write your pallas kernel here")


# speedup vs baseline: 1.7871x; 1.7871x over previous
import functools

import jax
import jax.numpy as jnp
from jax import lax
from jax.experimental import pallas as pl
from jax.experimental.pallas import tpu as pltpu
from jax.experimental.pallas import tpu_sc as plsc

BATCH = 4
SEQ = 4096
TOTAL = BATCH * SEQ
DIM = 1024
PAD = 1

NC = 2
NS = 16
NW = NC * NS
BPW = TOTAL // NW
CHUNK = 64
NCHUNK = BPW // CHUNK


def _pos_ids_body(ids_ref, out_ref):
    ids = ids_ref[...]
    m = (ids != PAD).astype(jnp.int32)
    c = m
    k = 1
    while k < SEQ:
        shifted = jnp.concatenate(
            [jnp.zeros((BATCH, k), jnp.int32), c[:, : SEQ - k]], axis=1
        )
        c = c + shifted
        k *= 2
    out_ref[...] = c * m + PAD


def _position_ids(input_ids):
    return pl.pallas_call(
        _pos_ids_body,
        out_shape=jax.ShapeDtypeStruct((BATCH, SEQ), jnp.int32),
    )(input_ids)


_sc_mesh = plsc.VectorSubcoreMesh(core_axis_name="c", subcore_axis_name="s")


@functools.partial(
    pl.kernel,
    mesh=_sc_mesh,
    out_type=jax.ShapeDtypeStruct((TOTAL, DIM), jnp.float32),
    scratch_types=[
        pltpu.VMEM((NCHUNK, CHUNK), jnp.int32),
        pltpu.VMEM((CHUNK, DIM), jnp.float32),
        pltpu.SemaphoreType.DMA,
    ],
)
def _sc_gather(table_hbm, idx_hbm, out_hbm, idx_v, rows_v, sem):
    wid = lax.axis_index("s") * NC + lax.axis_index("c")
    base = wid * BPW
    pltpu.sync_copy(idx_hbm.at[wid], idx_v)
    for c in range(NCHUNK):
        pltpu.async_copy(table_hbm.at[idx_v.at[c]], rows_v, sem).wait()
        pltpu.sync_copy(rows_v, out_hbm.at[pl.ds(base + c * CHUNK, CHUNK)])


def kernel(input_ids, weights):
    pos = _position_ids(input_ids)
    idx = pos.reshape(NW, NCHUNK, CHUNK)
    out = _sc_gather(weights, idx)
    return out.reshape(BATCH, SEQ, weights.shape[-1])


# speedup vs baseline: 1.9418x; 1.0866x over previous
import functools

import jax
import jax.numpy as jnp
from jax import lax
from jax.experimental import pallas as pl
from jax.experimental.pallas import tpu as pltpu
from jax.experimental.pallas import tpu_sc as plsc

BATCH = 4
SEQ = 4096
TOTAL = BATCH * SEQ
DIM = 1024
PAD = 1

NC = 2
NS = 16
NW = NC * NS
BPW = TOTAL // NW
CHUNK = 32
NCHUNK = BPW // CHUNK
NB = 3


def _pos_ids_body(ids_ref, out_ref):
    ids = ids_ref[...]
    m = (ids != PAD).astype(jnp.int32)
    c = m
    k = 1
    while k < SEQ:
        shifted = jnp.concatenate(
            [jnp.zeros((BATCH, k), jnp.int32), c[:, : SEQ - k]], axis=1
        )
        c = c + shifted
        k *= 2
    out_ref[...] = c * m + PAD


def _position_ids(input_ids):
    return pl.pallas_call(
        _pos_ids_body,
        out_shape=jax.ShapeDtypeStruct((BATCH, SEQ), jnp.int32),
    )(input_ids)


_sc_mesh = plsc.VectorSubcoreMesh(core_axis_name="c", subcore_axis_name="s")


@functools.partial(
    pl.kernel,
    mesh=_sc_mesh,
    out_type=jax.ShapeDtypeStruct((TOTAL, DIM), jnp.float32),
    scratch_types=[
        pltpu.VMEM((NCHUNK, CHUNK), jnp.int32),
        pltpu.VMEM((CHUNK, DIM), jnp.float32),
        pltpu.VMEM((CHUNK, DIM), jnp.float32),
        pltpu.VMEM((CHUNK, DIM), jnp.float32),
        pltpu.SemaphoreType.DMA,
        pltpu.SemaphoreType.DMA,
        pltpu.SemaphoreType.DMA,
        pltpu.SemaphoreType.DMA,
        pltpu.SemaphoreType.DMA,
        pltpu.SemaphoreType.DMA,
    ],
)
def _sc_gather(table_hbm, idx_hbm, out_hbm, idx_v,
               b0, b1, b2, gs0, gs1, gs2, ps0, ps1, ps2):
    bufs = (b0, b1, b2)
    gsems = (gs0, gs1, gs2)
    psems = (ps0, ps1, ps2)
    wid = lax.axis_index("s") * NC + lax.axis_index("c")
    base = wid * BPW
    pltpu.sync_copy(idx_hbm.at[wid], idx_v)
    gets = [None] * NCHUNK
    puts = [None] * NCHUNK

    def _put(c):
        b = c % NB
        return pltpu.async_copy(
            bufs[b], out_hbm.at[pl.ds(base + c * CHUNK, CHUNK)], psems[b]
        )

    for c in range(NCHUNK):
        b = c % NB
        if c >= NB:
            puts[c - NB].wait()
        gets[c] = pltpu.async_copy(table_hbm.at[idx_v.at[c]], bufs[b], gsems[b])
        if c >= 1:
            gets[c - 1].wait()
            puts[c - 1] = _put(c - 1)
    gets[NCHUNK - 1].wait()
    puts[NCHUNK - 1] = _put(NCHUNK - 1)
    for c in range(NCHUNK - NB, NCHUNK):
        puts[c].wait()


def kernel(input_ids, weights):
    pos = _position_ids(input_ids)
    idx = pos.reshape(NW, NCHUNK, CHUNK)
    out = _sc_gather(weights, idx)
    return out.reshape(BATCH, SEQ, weights.shape[-1])
